# baseline (speedup 1.0000x reference)
import functools

import jax
import jax.numpy as jnp
from jax import lax
from jax.experimental import pallas as pl
from jax.experimental.pallas import tpu as pltpu
from jax.experimental.pallas import tpu_sc as plsc

N = 10000
D_IN = 128
D_MID = 256
D_FIN = 128
E = 320000

LANES = 16
NC = 2
NS = 16

EPR = 128
ROWS_PAD = 2560
E_PAD = ROWS_PAD * EPR
RPT = ROWS_PAD // (NC * NS)
IDXC = 40
ACC_ROWS = 10008
NPT = N // NS
DEG_EPT = E_PAD // (NC * NS)
CNT_ROWS = 10016

BN = 2000
GRID = N // BN

_MESH = plsc.VectorSubcoreMesh(
    core_axis_name="c", subcore_axis_name="s", num_cores=NC, num_subcores=NS)


def _deg_body(dst_hbm, out_hbm, dst_v, cnt_v):
    c = lax.axis_index("c")
    s = lax.axis_index("s")
    wid = s * NC + c
    pltpu.sync_copy(dst_hbm.at[pl.ds(wid * DEG_EPT, DEG_EPT)], dst_v)
    zeros16 = jnp.zeros((LANES,), jnp.float32)
    ones16 = jnp.ones((LANES,), jnp.float32)

    def zloop(i, carry):
        cnt_v[pl.ds(i * LANES, LANES)] = zeros16
        return carry

    lax.fori_loop(0, CNT_ROWS // LANES, zloop, 0)

    def aloop(j, carry):
        idx = dst_v[pl.ds(j * LANES, LANES)]
        plsc.addupdate_scatter(cnt_v, [idx], ones16)
        return carry

    lax.fori_loop(0, DEG_EPT // LANES, aloop, 0)
    pltpu.sync_copy(cnt_v.at[pl.ds(0, N)], out_hbm.at[wid])


_deg_call = pl.kernel(
    _deg_body,
    out_type=jax.ShapeDtypeStruct((NC * NS, N), jnp.float32),
    mesh=_MESH,
    compiler_params=pltpu.CompilerParams(
        needs_layout_passes=False, use_tc_tiling_on_sc=False),
    scratch_types=[
        pltpu.VMEM((DEG_EPT,), jnp.int32),
        pltpu.VMEM((CNT_ROWS,), jnp.float32),
    ],
)


def _make_agg(d_full):
    def body(g_hbm, src_hbm, dst_hbm, zeros_hbm, out_hbm,
             idx_s, idx_d, buf0, buf1, acc_sh, gs0, gs1):
        c = lax.axis_index("c")
        s = lax.axis_index("s")
        bufs = (buf0, buf1)
        gsems = (gs0, gs1)
        pltpu.sync_copy(zeros_hbm, acc_sh.at[pl.ds(s * NPT, NPT)])
        plsc.subcore_barrier()

        def gstart(j, b):
            pltpu.async_copy(g_hbm.at[idx_s.at[j]], bufs[b], gsems[b])

        def gwait(b):
            pltpu.make_async_copy(g_hbm.at[idx_s.at[0]], bufs[b],
                                  gsems[b]).wait()

        def scat(j, b):
            pltpu.sync_copy(bufs[b], acc_sh.at[idx_d.at[j]], add=True)

        def chunk(ci, carry):
            row0 = c * (ROWS_PAD // NC) + s * RPT + ci * IDXC
            pltpu.sync_copy(src_hbm.at[pl.ds(row0, IDXC)], idx_s)
            pltpu.sync_copy(dst_hbm.at[pl.ds(row0, IDXC)], idx_d)
            gstart(0, 0)

            def pair(j, c2):
                r = 2 * j
                gstart(r + 1, 1)
                gwait(0)
                scat(r, 0)

                @pl.when(j < IDXC // 2 - 1)
                def _():
                    gstart(r + 2, 0)

                gwait(1)
                scat(r + 1, 1)
                return c2

            lax.fori_loop(0, IDXC // 2, pair, carry)
            return carry

        lax.fori_loop(0, RPT // IDXC, chunk, 0)
        plsc.subcore_barrier()
        pltpu.sync_copy(
            acc_sh.at[pl.ds(s * NPT, NPT)],
            out_hbm.at[pl.ds(c * N + s * NPT, NPT)])

    return pl.kernel(
        body,
        out_type=jax.ShapeDtypeStruct((NC * N, d_full), jnp.bfloat16),
        mesh=_MESH,
        compiler_params=pltpu.CompilerParams(use_tc_tiling_on_sc=False),
        scratch_types=(
            [pltpu.VMEM((IDXC, EPR), jnp.int32),
             pltpu.VMEM((IDXC, EPR), jnp.int32)]
            + [pltpu.VMEM((EPR, d_full), jnp.bfloat16)] * 2
            + [pltpu.VMEM_SHARED((ACC_ROWS, d_full), jnp.bfloat16)]
            + [pltpu.SemaphoreType.DMA] * 2
        ),
    )


_agg_l1 = _make_agg(D_MID)
_agg_l2 = _make_agg(D_FIN)


def _tc_a_body(x_ref, w_ref, degp_ref, g_ref, dinv_ref):
    deg = jnp.sum(degp_ref[...], axis=1, keepdims=True) + 1.0
    dinv = lax.rsqrt(deg)
    h = jnp.dot(x_ref[...], w_ref[...], preferred_element_type=jnp.float32)
    g_ref[...] = (h * dinv).astype(jnp.bfloat16)
    dinv_ref[...] = dinv


_tc_a = pl.pallas_call(
    _tc_a_body,
    grid=(GRID,),
    in_specs=[
        pl.BlockSpec((BN, D_IN), lambda i: (i, 0)),
        pl.BlockSpec((D_IN, D_MID), lambda i: (0, 0)),
        pl.BlockSpec((BN, NC * NS), lambda i: (i, 0)),
    ],
    out_specs=[
        pl.BlockSpec((BN, D_MID), lambda i: (i, 0)),
        pl.BlockSpec((BN, 1), lambda i: (i, 0)),
    ],
    out_shape=[
        jax.ShapeDtypeStruct((N, D_MID), jnp.bfloat16),
        jax.ShapeDtypeStruct((N, 1), jnp.float32),
    ],
)


def _tc_b_body(acca_ref, accb_ref, g_ref, dinv_ref, b1_ref, w2_ref,
               g2_ref):
    acc = acca_ref[...].astype(jnp.float32) + accb_ref[...].astype(jnp.float32)
    g = g_ref[...].astype(jnp.float32)
    dinv = dinv_ref[...]
    h1 = jnp.maximum((acc + g) * dinv + b1_ref[...][None, :], 0.0)
    h2 = jnp.dot(h1, w2_ref[...], preferred_element_type=jnp.float32)
    g2_ref[...] = (h2 * dinv).astype(jnp.bfloat16)


_tc_b = pl.pallas_call(
    _tc_b_body,
    grid=(GRID,),
    in_specs=[
        pl.BlockSpec((BN, D_MID), lambda i: (i, 0)),
        pl.BlockSpec((BN, D_MID), lambda i: (i + GRID, 0)),
        pl.BlockSpec((BN, D_MID), lambda i: (i, 0)),
        pl.BlockSpec((BN, 1), lambda i: (i, 0)),
        pl.BlockSpec((D_MID,), lambda i: (0,)),
        pl.BlockSpec((D_MID, D_FIN), lambda i: (0, 0)),
    ],
    out_specs=pl.BlockSpec((BN, D_FIN), lambda i: (i, 0)),
    out_shape=jax.ShapeDtypeStruct((N, D_FIN), jnp.bfloat16),
)


def _tc_c_body(acca_ref, accb_ref, g_ref, dinv_ref, b2_ref, out_ref):
    acc = acca_ref[...].astype(jnp.float32) + accb_ref[...].astype(jnp.float32)
    g = g_ref[...].astype(jnp.float32)
    out = (acc + g) * dinv_ref[...] + b2_ref[...][None, :]
    out_ref[...] = jnp.maximum(out, 0.0)


_tc_c = pl.pallas_call(
    _tc_c_body,
    grid=(GRID,),
    in_specs=[
        pl.BlockSpec((BN, D_FIN), lambda i: (i, 0)),
        pl.BlockSpec((BN, D_FIN), lambda i: (i + GRID, 0)),
        pl.BlockSpec((BN, D_FIN), lambda i: (i, 0)),
        pl.BlockSpec((BN, 1), lambda i: (i, 0)),
        pl.BlockSpec((D_FIN,), lambda i: (0,)),
    ],
    out_specs=pl.BlockSpec((BN, D_FIN), lambda i: (i, 0)),
    out_shape=jax.ShapeDtypeStruct((N, D_FIN), jnp.float32),
)


@jax.jit
def kernel(x, edge_index, W1, b1, W2, b2):
    src = edge_index[0]
    dst = edge_index[1]
    npad = E_PAD - E
    src_pad = jnp.concatenate([src, jnp.zeros((npad,), jnp.int32)])
    dst_pad = jnp.concatenate([dst, jnp.full((npad,), N, jnp.int32)])
    src2d = src_pad.reshape(ROWS_PAD, EPR)
    dst2d = dst_pad.reshape(ROWS_PAD, EPR)
    z1 = jnp.zeros((NPT, D_MID), jnp.bfloat16)
    z2 = jnp.zeros((NPT, D_FIN), jnp.bfloat16)

    degp = _deg_call(dst_pad)
    g1, dinv = _tc_a(x, W1, degp.T)
    acc1 = _agg_l1(g1, src2d, dst2d, z1)
    g2 = _tc_b(acc1, acc1, g1, dinv, b1, W2)
    acc2 = _agg_l2(g2, src2d, dst2d, z2)
    return _tc_c(acc2, acc2, g2, dinv, b2)


# speedup vs baseline: 1.4151x; 1.4151x over previous
import functools

import jax
import jax.numpy as jnp
from jax import lax
from jax.experimental import pallas as pl
from jax.experimental.pallas import tpu as pltpu
from jax.experimental.pallas import tpu_sc as plsc

N = 10000
D_IN = 128
D_MID = 256
D_FIN = 128
E = 320000

LANES = 16
NC = 2
NS = 16

EPR = 128
ROWS_PAD = 2560
E_PAD = ROWS_PAD * EPR
RPT = ROWS_PAD // NS
IDXC = 40
UPS = 2
ACC_ROWS = 10008
NPT = N // NS
DEG_EPT = E_PAD // (NC * NS)
CNT_ROWS = 10016

BN = 2000
GRID = N // BN

_MESH = plsc.VectorSubcoreMesh(
    core_axis_name="c", subcore_axis_name="s", num_cores=NC, num_subcores=NS)


def _deg_body(dst_hbm, out_hbm, dst_v, cnt_v):
    c = lax.axis_index("c")
    s = lax.axis_index("s")
    wid = s * NC + c
    pltpu.sync_copy(dst_hbm.at[pl.ds(wid * DEG_EPT, DEG_EPT)], dst_v)
    zeros16 = jnp.zeros((LANES,), jnp.float32)
    ones16 = jnp.ones((LANES,), jnp.float32)

    def zloop(i, carry):
        cnt_v[pl.ds(i * LANES, LANES)] = zeros16
        return carry

    lax.fori_loop(0, CNT_ROWS // LANES, zloop, 0)

    def aloop(j, carry):
        idx = dst_v[pl.ds(j * LANES, LANES)]
        plsc.addupdate_scatter(cnt_v, [idx], ones16)
        return carry

    lax.fori_loop(0, DEG_EPT // LANES, aloop, 0)
    pltpu.sync_copy(cnt_v.at[pl.ds(0, N)], out_hbm.at[wid])


_deg_call = pl.kernel(
    _deg_body,
    out_type=jax.ShapeDtypeStruct((NC * NS, N), jnp.float32),
    mesh=_MESH,
    compiler_params=pltpu.CompilerParams(
        needs_layout_passes=False, use_tc_tiling_on_sc=False),
    scratch_types=[
        pltpu.VMEM((DEG_EPT,), jnp.int32),
        pltpu.VMEM((CNT_ROWS,), jnp.float32),
    ],
)


def _make_agg(d_half, dtype=jnp.bfloat16):
    def body(glo, ghi, src_hbm, dst_hbm, zeros_hbm, out_hbm,
             idx_s, idx_d, buf0, buf1, buf2, buf3, acc_sh,
             gs0, gs1, gs2, gs3, ss0, ss1, ss2, ss3):
        c = lax.axis_index("c")
        s = lax.axis_index("s")
        bufs = (buf0, buf1, buf2, buf3)
        gsems = (gs0, gs1, gs2, gs3)
        ssems = (ss0, ss1, ss2, ss3)
        pltpu.sync_copy(zeros_hbm, acc_sh.at[pl.ds(s * NPT, NPT)])
        plsc.subcore_barrier()

        def gstart(u, b):
            ix = idx_s.at[pl.ds(UPS * EPR * u, UPS * EPR)]

            @pl.when(c == 0)
            def _():
                pltpu.async_copy(glo.at[ix], bufs[b], gsems[b])

            @pl.when(c == 1)
            def _():
                pltpu.async_copy(ghi.at[ix], bufs[b], gsems[b])

        def gwait(b):
            pltpu.make_async_copy(glo.at[idx_s.at[pl.ds(0, UPS * EPR)]],
                                  bufs[b], gsems[b]).wait()

        def sstart(u, b):
            pltpu.async_copy(
                bufs[b], acc_sh.at[idx_d.at[pl.ds(UPS * EPR * u, UPS * EPR)]],
                ssems[b], add=True)

        def swait(b):
            pltpu.make_async_copy(bufs[b],
                                  acc_sh.at[idx_d.at[pl.ds(0, UPS * EPR)]],
                                  ssems[b]).wait()

        UC = IDXC // UPS

        def chunk(ci, carry):
            e0 = (s * RPT + ci * IDXC) * EPR
            pltpu.sync_copy(src_hbm.at[pl.ds(e0, IDXC * EPR)], idx_s)
            pltpu.sync_copy(dst_hbm.at[pl.ds(e0, IDXC * EPR)], idx_d)
            gstart(0, 0)
            gstart(1, 1)
            gstart(2, 2)
            gwait(0)
            sstart(0, 0)
            gstart(3, 3)
            gwait(1)
            sstart(1, 1)
            swait(0)
            gstart(4, 0)
            gwait(2)
            sstart(2, 2)
            swait(1)
            gstart(5, 1)
            gwait(3)
            sstart(3, 3)

            def group(g, c2):
                base = 4 * g
                for b in range(4):
                    u = base + b
                    bn = (b + 2) % 4
                    swait(bn)
                    gstart(u + 2, bn)
                    gwait(b)
                    sstart(u, b)
                return c2

            lax.fori_loop(1, UC // 4 - 1, group, carry)
            swait(2)
            gstart(UC - 2, 2)
            gwait(0)
            sstart(UC - 4, 0)
            swait(3)
            gstart(UC - 1, 3)
            gwait(1)
            sstart(UC - 3, 1)
            gwait(2)
            sstart(UC - 2, 2)
            gwait(3)
            sstart(UC - 1, 3)
            for b in range(4):
                swait(b)
            return carry

        lax.fori_loop(0, RPT // IDXC, chunk, 0)
        plsc.subcore_barrier()
        pltpu.sync_copy(
            acc_sh.at[pl.ds(s * NPT, NPT)],
            out_hbm.at[pl.ds(s * NPT, NPT), pl.ds(c * d_half, d_half)])

    return pl.kernel(
        body,
        out_type=jax.ShapeDtypeStruct((N, 2 * d_half), dtype),
        mesh=_MESH,
        compiler_params=pltpu.CompilerParams(use_tc_tiling_on_sc=False),
        scratch_types=(
            [pltpu.VMEM((IDXC * EPR,), jnp.int32),
             pltpu.VMEM((IDXC * EPR,), jnp.int32)]
            + [pltpu.VMEM((UPS * EPR, d_half), dtype)] * 4
            + [pltpu.VMEM_SHARED((ACC_ROWS, d_half), dtype)]
            + [pltpu.SemaphoreType.DMA] * 8
        ),
    )


_agg_l1 = _make_agg(128)
_agg_l2 = _make_agg(64)


def _tc_a_body(x_ref, w_ref, degp_ref, glo_ref, ghi_ref, dinv_ref):
    deg = jnp.sum(degp_ref[...], axis=1, keepdims=True) + 1.0
    dinv = lax.rsqrt(deg)
    h = jnp.dot(x_ref[...], w_ref[...], preferred_element_type=jnp.float32)
    g = (h * dinv).astype(jnp.bfloat16)
    glo_ref[...] = g[:, :D_MID // 2]
    ghi_ref[...] = g[:, D_MID // 2:]
    dinv_ref[...] = dinv


_tc_a = pl.pallas_call(
    _tc_a_body,
    grid=(GRID,),
    in_specs=[
        pl.BlockSpec((BN, D_IN), lambda i: (i, 0)),
        pl.BlockSpec((D_IN, D_MID), lambda i: (0, 0)),
        pl.BlockSpec((BN, NC * NS), lambda i: (i, 0)),
    ],
    out_specs=[
        pl.BlockSpec((BN, D_MID // 2), lambda i: (i, 0)),
        pl.BlockSpec((BN, D_MID // 2), lambda i: (i, 0)),
        pl.BlockSpec((BN, 1), lambda i: (i, 0)),
    ],
    out_shape=[
        jax.ShapeDtypeStruct((N, D_MID // 2), jnp.bfloat16),
        jax.ShapeDtypeStruct((N, D_MID // 2), jnp.bfloat16),
        jax.ShapeDtypeStruct((N, 1), jnp.float32),
    ],
)


def _tc_b_body(acc_ref, glo_ref, ghi_ref, dinv_ref, b1_ref, w2_ref,
               g2lo_ref, g2hi_ref):
    g = jnp.concatenate([glo_ref[...], ghi_ref[...]],
                        axis=1).astype(jnp.float32)
    acc = acc_ref[...].astype(jnp.float32)
    dinv = dinv_ref[...]
    h1 = jnp.maximum((acc + g) * dinv + b1_ref[...][None, :], 0.0)
    h2 = jnp.dot(h1, w2_ref[...], preferred_element_type=jnp.float32)
    g2 = (h2 * dinv).astype(jnp.bfloat16)
    g2lo_ref[...] = g2[:, :D_FIN // 2]
    g2hi_ref[...] = g2[:, D_FIN // 2:]


_tc_b = pl.pallas_call(
    _tc_b_body,
    grid=(GRID,),
    in_specs=[
        pl.BlockSpec((BN, D_MID), lambda i: (i, 0)),
        pl.BlockSpec((BN, D_MID // 2), lambda i: (i, 0)),
        pl.BlockSpec((BN, D_MID // 2), lambda i: (i, 0)),
        pl.BlockSpec((BN, 1), lambda i: (i, 0)),
        pl.BlockSpec((D_MID,), lambda i: (0,)),
        pl.BlockSpec((D_MID, D_FIN), lambda i: (0, 0)),
    ],
    out_specs=[
        pl.BlockSpec((BN, D_FIN // 2), lambda i: (i, 0)),
        pl.BlockSpec((BN, D_FIN // 2), lambda i: (i, 0)),
    ],
    out_shape=[
        jax.ShapeDtypeStruct((N, D_FIN // 2), jnp.bfloat16),
        jax.ShapeDtypeStruct((N, D_FIN // 2), jnp.bfloat16),
    ],
)


def _tc_c_body(acc_ref, glo_ref, ghi_ref, dinv_ref, b2_ref, out_ref):
    g = jnp.concatenate([glo_ref[...], ghi_ref[...]],
                        axis=1).astype(jnp.float32)
    acc = acc_ref[...].astype(jnp.float32)
    out = (acc + g) * dinv_ref[...] + b2_ref[...][None, :]
    out_ref[...] = jnp.maximum(out, 0.0)


_tc_c = pl.pallas_call(
    _tc_c_body,
    grid=(GRID,),
    in_specs=[
        pl.BlockSpec((BN, D_FIN), lambda i: (i, 0)),
        pl.BlockSpec((BN, D_FIN // 2), lambda i: (i, 0)),
        pl.BlockSpec((BN, D_FIN // 2), lambda i: (i, 0)),
        pl.BlockSpec((BN, 1), lambda i: (i, 0)),
        pl.BlockSpec((D_FIN,), lambda i: (0,)),
    ],
    out_specs=pl.BlockSpec((BN, D_FIN), lambda i: (i, 0)),
    out_shape=jax.ShapeDtypeStruct((N, D_FIN), jnp.float32),
)


@jax.jit
def kernel(x, edge_index, W1, b1, W2, b2):
    src = edge_index[0]
    dst = edge_index[1]
    npad = E_PAD - E
    src_pad = jnp.concatenate([src, jnp.zeros((npad,), jnp.int32)])
    dst_pad = jnp.concatenate([dst, jnp.full((npad,), N, jnp.int32)])
    z1 = jnp.zeros((NPT, 128), jnp.bfloat16)
    z2 = jnp.zeros((NPT, 64), jnp.bfloat16)

    degp = _deg_call(dst_pad)
    g1lo, g1hi, dinv = _tc_a(x, W1, degp.T)
    acc1 = _agg_l1(g1lo, g1hi, src_pad, dst_pad, z1)
    g2lo, g2hi = _tc_b(acc1, g1lo, g1hi, dinv, b1, W2)
    acc2 = _agg_l2(g2lo, g2hi, src_pad, dst_pad, z2)
    return _tc_c(acc2, g2lo, g2hi, dinv, b2)


# speedup vs baseline: 1.4465x; 1.0222x over previous
import functools

import jax
import jax.numpy as jnp
from jax import lax
from jax.experimental import pallas as pl
from jax.experimental.pallas import tpu as pltpu
from jax.experimental.pallas import tpu_sc as plsc

N = 10000
D_IN = 128
D_MID = 256
D_FIN = 128
E = 320000

LANES = 16
NC = 2
NS = 16

EPR = 128
ROWS_PAD = 2560
E_PAD = ROWS_PAD * EPR
RPT = ROWS_PAD // NS
IDXC = 32
ACC_ROWS = 10008
NPT = N // NS
DEG_EPT = E_PAD // (NC * NS)
CNT_ROWS = 10016

BN = 2000
GRID = N // BN

_MESH = plsc.VectorSubcoreMesh(
    core_axis_name="c", subcore_axis_name="s", num_cores=NC, num_subcores=NS)


def _deg_body(dst_hbm, out_hbm, dst_v, cnt_v):
    c = lax.axis_index("c")
    s = lax.axis_index("s")
    wid = s * NC + c
    pltpu.sync_copy(dst_hbm.at[pl.ds(wid * DEG_EPT, DEG_EPT)], dst_v)
    zeros16 = jnp.zeros((LANES,), jnp.float32)
    ones16 = jnp.ones((LANES,), jnp.float32)

    def zloop(i, carry):
        cnt_v[pl.ds(i * LANES, LANES)] = zeros16
        return carry

    lax.fori_loop(0, CNT_ROWS // LANES, zloop, 0)

    def aloop(j, carry):
        idx = dst_v[pl.ds(j * LANES, LANES)]
        plsc.addupdate_scatter(cnt_v, [idx], ones16)
        return carry

    lax.fori_loop(0, DEG_EPT // LANES, aloop, 0)
    pltpu.sync_copy(cnt_v.at[pl.ds(0, N)], out_hbm.at[wid])


_deg_call = pl.kernel(
    _deg_body,
    out_type=jax.ShapeDtypeStruct((NC * NS, N), jnp.float32),
    mesh=_MESH,
    compiler_params=pltpu.CompilerParams(
        needs_layout_passes=False, use_tc_tiling_on_sc=False),
    scratch_types=[
        pltpu.VMEM((DEG_EPT,), jnp.int32),
        pltpu.VMEM((CNT_ROWS,), jnp.float32),
    ],
)


def _make_agg(d_half, dtype=jnp.bfloat16):
    def body(glo, ghi, src_hbm, dst_hbm, zeros_hbm, out_hbm,
             idx_s, idx_d, buf0, buf1, buf2, buf3, acc_sh,
             gs0, gs1, gs2, gs3, ss0, ss1, ss2, ss3):
        c = lax.axis_index("c")
        s = lax.axis_index("s")
        bufs = (buf0, buf1, buf2, buf3)
        gsems = (gs0, gs1, gs2, gs3)
        ssems = (ss0, ss1, ss2, ss3)
        pltpu.sync_copy(zeros_hbm, acc_sh.at[pl.ds(s * NPT, NPT)])
        pltpu.sync_copy(src_hbm.at[pl.ds(s * RPT, RPT)], idx_s)
        pltpu.sync_copy(dst_hbm.at[pl.ds(s * RPT, RPT)], idx_d)
        plsc.subcore_barrier()

        def gstart(j, b):
            @pl.when(c == 0)
            def _():
                pltpu.async_copy(glo.at[idx_s.at[j]], bufs[b], gsems[b])

            @pl.when(c == 1)
            def _():
                pltpu.async_copy(ghi.at[idx_s.at[j]], bufs[b], gsems[b])

        def gwait(b):
            pltpu.make_async_copy(glo.at[idx_s.at[0]], bufs[b], gsems[b]).wait()

        def sstart(j, b):
            pltpu.async_copy(bufs[b], acc_sh.at[idx_d.at[j]], ssems[b],
                             add=True)

        def swait(b):
            pltpu.make_async_copy(bufs[b], acc_sh.at[idx_d.at[0]],
                                  ssems[b]).wait()

        gstart(0, 0)
        gstart(1, 1)
        gstart(2, 2)
        gwait(0)
        sstart(0, 0)
        gstart(3, 3)
        gwait(1)
        sstart(1, 1)
        swait(0)
        gstart(4, 0)
        gwait(2)
        sstart(2, 2)
        swait(1)
        gstart(5, 1)
        gwait(3)
        sstart(3, 3)

        def group(g, carry):
            base = 4 * g
            for b in range(4):
                j = base + b
                bn = (b + 2) % 4
                swait(bn)
                gstart(j + 2, bn)
                gwait(b)
                sstart(j, b)
            return carry

        lax.fori_loop(1, RPT // 4 - 1, group, 0)
        swait(2)
        gstart(RPT - 2, 2)
        gwait(0)
        sstart(RPT - 4, 0)
        swait(3)
        gstart(RPT - 1, 3)
        gwait(1)
        sstart(RPT - 3, 1)
        gwait(2)
        sstart(RPT - 2, 2)
        gwait(3)
        sstart(RPT - 1, 3)
        for b in range(4):
            swait(b)
        plsc.subcore_barrier()
        pltpu.sync_copy(
            acc_sh.at[pl.ds(s * NPT, NPT)],
            out_hbm.at[pl.ds(s * NPT, NPT), pl.ds(c * d_half, d_half)])

    return pl.kernel(
        body,
        out_type=jax.ShapeDtypeStruct((N, 2 * d_half), dtype),
        mesh=_MESH,
        compiler_params=pltpu.CompilerParams(use_tc_tiling_on_sc=False),
        scratch_types=(
            [pltpu.VMEM((RPT, EPR), jnp.int32),
             pltpu.VMEM((RPT, EPR), jnp.int32)]
            + [pltpu.VMEM((EPR, d_half), dtype)] * 4
            + [pltpu.VMEM_SHARED((ACC_ROWS, d_half), dtype)]
            + [pltpu.SemaphoreType.DMA] * 8
        ),
    )


_agg_l1 = _make_agg(128)
_agg_l2 = _make_agg(64)


def _tc_a_body(x_ref, w_ref, degp_ref, glo_ref, ghi_ref, dinv_ref):
    deg = jnp.sum(degp_ref[...], axis=1, keepdims=True) + 1.0
    dinv = lax.rsqrt(deg)
    h = jnp.dot(x_ref[...], w_ref[...], preferred_element_type=jnp.float32)
    g = (h * dinv).astype(jnp.bfloat16)
    glo_ref[...] = g[:, :D_MID // 2]
    ghi_ref[...] = g[:, D_MID // 2:]
    dinv_ref[...] = dinv


_tc_a = pl.pallas_call(
    _tc_a_body,
    grid=(GRID,),
    in_specs=[
        pl.BlockSpec((BN, D_IN), lambda i: (i, 0)),
        pl.BlockSpec((D_IN, D_MID), lambda i: (0, 0)),
        pl.BlockSpec((BN, NC * NS), lambda i: (i, 0)),
    ],
    out_specs=[
        pl.BlockSpec((BN, D_MID // 2), lambda i: (i, 0)),
        pl.BlockSpec((BN, D_MID // 2), lambda i: (i, 0)),
        pl.BlockSpec((BN, 1), lambda i: (i, 0)),
    ],
    out_shape=[
        jax.ShapeDtypeStruct((N, D_MID // 2), jnp.bfloat16),
        jax.ShapeDtypeStruct((N, D_MID // 2), jnp.bfloat16),
        jax.ShapeDtypeStruct((N, 1), jnp.float32),
    ],
)


def _tc_b_body(acc_ref, glo_ref, ghi_ref, dinv_ref, b1_ref, w2_ref,
               g2lo_ref, g2hi_ref):
    g = jnp.concatenate([glo_ref[...], ghi_ref[...]],
                        axis=1).astype(jnp.float32)
    acc = acc_ref[...].astype(jnp.float32)
    dinv = dinv_ref[...]
    h1 = jnp.maximum((acc + g) * dinv + b1_ref[...][None, :], 0.0)
    h2 = jnp.dot(h1, w2_ref[...], preferred_element_type=jnp.float32)
    g2 = (h2 * dinv).astype(jnp.bfloat16)
    g2lo_ref[...] = g2[:, :D_FIN // 2]
    g2hi_ref[...] = g2[:, D_FIN // 2:]


_tc_b = pl.pallas_call(
    _tc_b_body,
    grid=(GRID,),
    in_specs=[
        pl.BlockSpec((BN, D_MID), lambda i: (i, 0)),
        pl.BlockSpec((BN, D_MID // 2), lambda i: (i, 0)),
        pl.BlockSpec((BN, D_MID // 2), lambda i: (i, 0)),
        pl.BlockSpec((BN, 1), lambda i: (i, 0)),
        pl.BlockSpec((D_MID,), lambda i: (0,)),
        pl.BlockSpec((D_MID, D_FIN), lambda i: (0, 0)),
    ],
    out_specs=[
        pl.BlockSpec((BN, D_FIN // 2), lambda i: (i, 0)),
        pl.BlockSpec((BN, D_FIN // 2), lambda i: (i, 0)),
    ],
    out_shape=[
        jax.ShapeDtypeStruct((N, D_FIN // 2), jnp.bfloat16),
        jax.ShapeDtypeStruct((N, D_FIN // 2), jnp.bfloat16),
    ],
)


def _tc_c_body(acc_ref, glo_ref, ghi_ref, dinv_ref, b2_ref, out_ref):
    g = jnp.concatenate([glo_ref[...], ghi_ref[...]],
                        axis=1).astype(jnp.float32)
    acc = acc_ref[...].astype(jnp.float32)
    out = (acc + g) * dinv_ref[...] + b2_ref[...][None, :]
    out_ref[...] = jnp.maximum(out, 0.0)


_tc_c = pl.pallas_call(
    _tc_c_body,
    grid=(GRID,),
    in_specs=[
        pl.BlockSpec((BN, D_FIN), lambda i: (i, 0)),
        pl.BlockSpec((BN, D_FIN // 2), lambda i: (i, 0)),
        pl.BlockSpec((BN, D_FIN // 2), lambda i: (i, 0)),
        pl.BlockSpec((BN, 1), lambda i: (i, 0)),
        pl.BlockSpec((D_FIN,), lambda i: (0,)),
    ],
    out_specs=pl.BlockSpec((BN, D_FIN), lambda i: (i, 0)),
    out_shape=jax.ShapeDtypeStruct((N, D_FIN), jnp.float32),
)


@jax.jit
def kernel(x, edge_index, W1, b1, W2, b2):
    src = edge_index[0]
    dst = edge_index[1]
    npad = E_PAD - E
    src_pad = jnp.concatenate([src, jnp.zeros((npad,), jnp.int32)])
    dst_pad = jnp.concatenate([dst, jnp.full((npad,), N, jnp.int32)])
    src2d = src_pad.reshape(ROWS_PAD, EPR)
    dst2d = dst_pad.reshape(ROWS_PAD, EPR)
    z1 = jnp.zeros((NPT, 128), jnp.bfloat16)
    z2 = jnp.zeros((NPT, 64), jnp.bfloat16)

    degp = _deg_call(dst_pad)
    g1lo, g1hi, dinv = _tc_a(x, W1, degp.T)
    acc1 = _agg_l1(g1lo, g1hi, src2d, dst2d, z1)
    g2lo, g2hi = _tc_b(acc1, g1lo, g1hi, dinv, b1, W2)
    acc2 = _agg_l2(g2lo, g2hi, src2d, dst2d, z2)
    return _tc_c(acc2, g2lo, g2hi, dinv, b2)


# speedup vs baseline: 1.9101x; 1.3205x over previous
import functools

import jax
import jax.numpy as jnp
from jax import lax
from jax.experimental import pallas as pl
from jax.experimental.pallas import tpu as pltpu
from jax.experimental.pallas import tpu_sc as plsc

N = 10000
D_IN = 128
D_MID = 256
D_FIN = 128
E = 320000

LANES = 16
NC = 2
NS = 16

EPR = 128
ROWS_PAD = 2560
E_PAD = ROWS_PAD * EPR
RPT = ROWS_PAD // NS
IDXC = 32
ACC_ROWS = 10008
NPT = N // NS
DEG_EPT = E_PAD // (NC * NS)
CNT_ROWS = 10016

BN = 2000
GRID = N // BN

_MESH = plsc.VectorSubcoreMesh(
    core_axis_name="c", subcore_axis_name="s", num_cores=NC, num_subcores=NS)


def _deg_body(dst_hbm, out_hbm, dst_v, cnt_v):
    c = lax.axis_index("c")
    s = lax.axis_index("s")
    wid = s * NC + c
    pltpu.sync_copy(dst_hbm.at[pl.ds(wid * DEG_EPT, DEG_EPT)], dst_v)
    zeros16 = jnp.zeros((LANES,), jnp.float32)
    ones16 = jnp.ones((LANES,), jnp.float32)

    def zloop(i, carry):
        cnt_v[pl.ds(i * LANES, LANES)] = zeros16
        return carry

    lax.fori_loop(0, CNT_ROWS // LANES, zloop, 0)

    def aloop(j, carry):
        idx = dst_v[pl.ds(j * LANES, LANES)]
        plsc.addupdate_scatter(cnt_v, [idx], ones16)
        return carry

    lax.fori_loop(0, DEG_EPT // LANES, aloop, 0)
    pltpu.sync_copy(cnt_v.at[pl.ds(0, N)], out_hbm.at[wid])


_deg_call = pl.kernel(
    _deg_body,
    out_type=jax.ShapeDtypeStruct((NC * NS, N), jnp.float32),
    mesh=_MESH,
    compiler_params=pltpu.CompilerParams(
        needs_layout_passes=False, use_tc_tiling_on_sc=False),
    scratch_types=[
        pltpu.VMEM((DEG_EPT,), jnp.int32),
        pltpu.VMEM((CNT_ROWS,), jnp.float32),
    ],
)


def _make_agg(d_half, dtype=jnp.bfloat16):
    def body(glo, ghi, src_hbm, dst_hbm, zeros_hbm, out_hbm,
             idx_s, idx_d, buf0, buf1, buf2, buf3, acc_sh,
             gs0, gs1, gs2, gs3, ss0, ss1, ss2, ss3):
        c = lax.axis_index("c")
        s = lax.axis_index("s")
        bufs = (buf0, buf1, buf2, buf3)
        gsems = (gs0, gs1, gs2, gs3)
        ssems = (ss0, ss1, ss2, ss3)
        pltpu.sync_copy(zeros_hbm, acc_sh.at[pl.ds(s * NPT, NPT)])
        pltpu.sync_copy(src_hbm.at[pl.ds(s * RPT, RPT)], idx_s)
        pltpu.sync_copy(dst_hbm.at[pl.ds(s * RPT, RPT)], idx_d)
        plsc.subcore_barrier()

        def gstart(j, b):
            @pl.when(c == 0)
            def _():
                pltpu.async_copy(glo.at[idx_s.at[j]], bufs[b], gsems[b])

            @pl.when(c == 1)
            def _():
                pltpu.async_copy(ghi.at[idx_s.at[j]], bufs[b], gsems[b])

        def gwait(b):
            pltpu.make_async_copy(glo.at[idx_s.at[0]], bufs[b], gsems[b]).wait()

        def sstart(j, b):
            pltpu.async_copy(bufs[b], acc_sh.at[idx_d.at[j]], ssems[b],
                             add=True)

        def swait(b):
            pltpu.make_async_copy(bufs[b], acc_sh.at[idx_d.at[0]],
                                  ssems[b]).wait()

        gstart(0, 0)
        gstart(1, 1)
        gstart(2, 2)
        gwait(0)
        sstart(0, 0)
        gstart(3, 3)
        gwait(1)
        sstart(1, 1)
        swait(0)
        gstart(4, 0)
        gwait(2)
        sstart(2, 2)
        swait(1)
        gstart(5, 1)
        gwait(3)
        sstart(3, 3)

        def group(g, carry):
            base = 4 * g
            for b in range(4):
                j = base + b
                bn = (b + 2) % 4
                swait(bn)
                gstart(j + 2, bn)
                gwait(b)
                sstart(j, b)
            return carry

        lax.fori_loop(1, RPT // 4 - 1, group, 0)
        swait(2)
        gstart(RPT - 2, 2)
        gwait(0)
        sstart(RPT - 4, 0)
        swait(3)
        gstart(RPT - 1, 3)
        gwait(1)
        sstart(RPT - 3, 1)
        gwait(2)
        sstart(RPT - 2, 2)
        gwait(3)
        sstart(RPT - 1, 3)
        for b in range(4):
            swait(b)
        plsc.subcore_barrier()
        pltpu.sync_copy(
            acc_sh.at[pl.ds(s * NPT, NPT)],
            out_hbm.at[pl.ds(s * NPT, NPT), pl.ds(c * d_half, d_half)])

    return pl.kernel(
        body,
        out_type=jax.ShapeDtypeStruct((N, 2 * d_half), dtype),
        mesh=_MESH,
        compiler_params=pltpu.CompilerParams(use_tc_tiling_on_sc=False),
        scratch_types=(
            [pltpu.VMEM((RPT, EPR), jnp.int32),
             pltpu.VMEM((RPT, EPR), jnp.int32)]
            + [pltpu.VMEM((EPR, d_half), dtype)] * 4
            + [pltpu.VMEM_SHARED((ACC_ROWS, d_half), dtype)]
            + [pltpu.SemaphoreType.DMA] * 8
        ),
    )


_agg_l1 = _make_agg(64)
_agg_l2 = _make_agg(64)


def _tc_a_body(x_ref, degp_ref, xlo_ref, xhi_ref, dinv_ref):
    deg = jnp.sum(degp_ref[...], axis=1, keepdims=True) + 1.0
    dinv = lax.rsqrt(deg)
    xs = (x_ref[...] * dinv).astype(jnp.bfloat16)
    xlo_ref[...] = xs[:, :D_IN // 2]
    xhi_ref[...] = xs[:, D_IN // 2:]
    dinv_ref[...] = dinv


_tc_a = pl.pallas_call(
    _tc_a_body,
    grid=(GRID,),
    in_specs=[
        pl.BlockSpec((BN, D_IN), lambda i: (i, 0)),
        pl.BlockSpec((BN, NC * NS), lambda i: (i, 0)),
    ],
    out_specs=[
        pl.BlockSpec((BN, D_IN // 2), lambda i: (i, 0)),
        pl.BlockSpec((BN, D_IN // 2), lambda i: (i, 0)),
        pl.BlockSpec((BN, 1), lambda i: (i, 0)),
    ],
    out_shape=[
        jax.ShapeDtypeStruct((N, D_IN // 2), jnp.bfloat16),
        jax.ShapeDtypeStruct((N, D_IN // 2), jnp.bfloat16),
        jax.ShapeDtypeStruct((N, 1), jnp.float32),
    ],
)


def _tc_b_body(accx_ref, xlo_ref, xhi_ref, dinv_ref, w1_ref, b1_ref,
               w2_ref, g2lo_ref, g2hi_ref):
    xs = jnp.concatenate([xlo_ref[...], xhi_ref[...]],
                         axis=1).astype(jnp.float32)
    dinv = dinv_ref[...]
    y = (accx_ref[...].astype(jnp.float32) + xs) * dinv
    h1 = jnp.maximum(
        jnp.dot(y, w1_ref[...], preferred_element_type=jnp.float32)
        + b1_ref[...][None, :], 0.0)
    h2 = jnp.dot(h1, w2_ref[...], preferred_element_type=jnp.float32)
    g2 = (h2 * dinv).astype(jnp.bfloat16)
    g2lo_ref[...] = g2[:, :D_FIN // 2]
    g2hi_ref[...] = g2[:, D_FIN // 2:]


_tc_b = pl.pallas_call(
    _tc_b_body,
    grid=(GRID,),
    in_specs=[
        pl.BlockSpec((BN, D_IN), lambda i: (i, 0)),
        pl.BlockSpec((BN, D_IN // 2), lambda i: (i, 0)),
        pl.BlockSpec((BN, D_IN // 2), lambda i: (i, 0)),
        pl.BlockSpec((BN, 1), lambda i: (i, 0)),
        pl.BlockSpec((D_IN, D_MID), lambda i: (0, 0)),
        pl.BlockSpec((D_MID,), lambda i: (0,)),
        pl.BlockSpec((D_MID, D_FIN), lambda i: (0, 0)),
    ],
    out_specs=[
        pl.BlockSpec((BN, D_FIN // 2), lambda i: (i, 0)),
        pl.BlockSpec((BN, D_FIN // 2), lambda i: (i, 0)),
    ],
    out_shape=[
        jax.ShapeDtypeStruct((N, D_FIN // 2), jnp.bfloat16),
        jax.ShapeDtypeStruct((N, D_FIN // 2), jnp.bfloat16),
    ],
)


def _tc_c_body(acc_ref, glo_ref, ghi_ref, dinv_ref, b2_ref, out_ref):
    g = jnp.concatenate([glo_ref[...], ghi_ref[...]],
                        axis=1).astype(jnp.float32)
    acc = acc_ref[...].astype(jnp.float32)
    out = (acc + g) * dinv_ref[...] + b2_ref[...][None, :]
    out_ref[...] = jnp.maximum(out, 0.0)


_tc_c = pl.pallas_call(
    _tc_c_body,
    grid=(GRID,),
    in_specs=[
        pl.BlockSpec((BN, D_FIN), lambda i: (i, 0)),
        pl.BlockSpec((BN, D_FIN // 2), lambda i: (i, 0)),
        pl.BlockSpec((BN, D_FIN // 2), lambda i: (i, 0)),
        pl.BlockSpec((BN, 1), lambda i: (i, 0)),
        pl.BlockSpec((D_FIN,), lambda i: (0,)),
    ],
    out_specs=pl.BlockSpec((BN, D_FIN), lambda i: (i, 0)),
    out_shape=jax.ShapeDtypeStruct((N, D_FIN), jnp.float32),
)


@jax.jit
def kernel(x, edge_index, W1, b1, W2, b2):
    src = edge_index[0]
    dst = edge_index[1]
    npad = E_PAD - E
    src_pad = jnp.concatenate([src, jnp.zeros((npad,), jnp.int32)])
    dst_pad = jnp.concatenate([dst, jnp.full((npad,), N, jnp.int32)])
    src2d = src_pad.reshape(ROWS_PAD, EPR)
    dst2d = dst_pad.reshape(ROWS_PAD, EPR)
    z = jnp.zeros((NPT, 64), jnp.bfloat16)

    degp = _deg_call(dst_pad)
    xlo, xhi, dinv = _tc_a(x, degp.T)
    acc1x = _agg_l1(xlo, xhi, src2d, dst2d, z)
    g2lo, g2hi = _tc_b(acc1x, xlo, xhi, dinv, W1, b1, W2)
    acc2 = _agg_l2(g2lo, g2hi, src2d, dst2d, z)
    return _tc_c(acc2, g2lo, g2hi, dinv, b2)
